# baseline (speedup 1.0000x reference)
import functools

import jax
import jax.numpy as jnp
from jax import lax
from jax.experimental import pallas as pl
from jax.experimental.pallas import tpu as pltpu
from jax.experimental.pallas import tpu_sc as plsc

BATCH = 16384
SEQ = 50
DIM = 64
NCLS = 2
VOCAB = 1000000

LANES = 16
NVEC = DIM // LANES

CH = 8
TOK = CH * SEQ

VBLK = 2048
HBLK = VBLK // 2


def _relayout_body(in_ref, out_ref):
  x = in_ref[...]
  out_ref[:, 0:DIM] = x[:, 0:HBLK].T
  out_ref[:, DIM:2 * DIM] = x[:, HBLK:VBLK].T


NBLK = (VOCAB + VBLK - 1) // VBLK
OUT_ROWS = NBLK * HBLK


@functools.cache
def _build_tc():
  return pl.pallas_call(
      _relayout_body,
      grid=(NBLK,),
      in_specs=[pl.BlockSpec((DIM, VBLK), lambda i: (0, i))],
      out_specs=pl.BlockSpec((HBLK, 2 * DIM), lambda i: (i, 0)),
      out_shape=jax.ShapeDtypeStruct((OUT_ROWS, 2 * DIM), jnp.float32),
  )


@functools.cache
def _build_sc():
  info = plsc.get_sparse_core_info()
  nw = info.num_cores * info.num_subcores
  sent_w = BATCH // nw
  tok_w = sent_w * SEQ
  row_w = tok_w * 2
  nch = sent_w // CH

  mesh = plsc.VectorSubcoreMesh(core_axis_name="c", subcore_axis_name="s")

  @functools.partial(
      pl.kernel,
      out_type=jax.ShapeDtypeStruct((BATCH * NCLS,), jnp.float32),
      mesh=mesh,
      compiler_params=pltpu.CompilerParams(
          needs_layout_passes=False, use_tc_tiling_on_sc=False),
      scratch_types=[
          pltpu.VMEM((row_w,), jnp.int32),
          pltpu.VMEM((2 * TOK, DIM // 2), jnp.float32),
          pltpu.VMEM((NCLS, DIM), jnp.float32),
          pltpu.VMEM((LANES,), jnp.float32),
          pltpu.VMEM((sent_w * NCLS,), jnp.float32),
          pltpu.SemaphoreType.DMA,
      ],
  )
  def fasttext_kernel(idx_hbm, table_hbm, fcw_hbm, fcb_hbm, out_hbm,
                      idx_v, rows_v, w_v, b_v, out_v, sem):
    wid = lax.axis_index("s") * info.num_cores + lax.axis_index("c")

    pltpu.sync_copy(idx_hbm.at[pl.ds(wid * row_w, row_w)], idx_v)
    pltpu.sync_copy(fcw_hbm, w_v)
    pltpu.sync_copy(fcb_hbm, b_v)

    w = [[w_v[c, pl.ds(LANES * j, LANES)] for j in range(NVEC)]
         for c in range(NCLS)]
    bias_vec = b_v[pl.ds(0, LANES)]
    lane = lax.iota(jnp.int32, LANES)
    inv_seq = jnp.float32(1.0 / SEQ)

    @pl.loop(0, nch)
    def _chunk(g):
      pltpu.async_copy(
          table_hbm.at[idx_v.at[pl.ds(g * 2 * TOK, 2 * TOK)]], rows_v, sem
      ).wait()
      outvec = jnp.zeros((LANES,), jnp.float32)
      for s in range(CH):
        base = 2 * s * SEQ
        acc = tuple(
            rows_v[base + (j // 2), pl.ds(LANES * (j % 2), LANES)]
            for j in range(NVEC)
        )

        def tok_body(t, carry, base=base):
          return tuple(
              carry[j]
              + rows_v[base + 2 * t + (j // 2), pl.ds(LANES * (j % 2), LANES)]
              for j in range(NVEC)
          )

        acc = lax.fori_loop(1, SEQ, tok_body, acc, unroll=7)
        for c in range(NCLS):
          prod = acc[0] * w[c][0]
          for j in range(1, NVEC):
            prod = prod + acc[j] * w[c][j]
          outvec = jnp.where(lane == (s * NCLS + c), jnp.sum(prod), outvec)
      out_v[pl.ds(g * LANES, LANES)] = outvec * inv_seq + bias_vec

    pltpu.sync_copy(out_v, out_hbm.at[pl.ds(wid * sent_w * NCLS,
                                            sent_w * NCLS)])

  return fasttext_kernel


def kernel(indices, emb_table, fc_w, fc_b):
  table_packed = _build_tc()(emb_table.T)
  table_rows = table_packed.reshape(4 * OUT_ROWS, DIM // 2)
  idx = indices.reshape(-1).astype(jnp.int32)
  r0 = (4 * 1024) * (idx // VBLK) + 4 * (idx % HBLK) \
      + 2 * ((idx % VBLK) // HBLK)
  idx2 = jnp.stack([r0, r0 + 1], axis=-1).reshape(-1)
  fcb_tiled = jnp.tile(fc_b.astype(jnp.float32), LANES // NCLS)
  out_flat = _build_sc()(idx2, table_rows, fc_w, fcb_tiled)
  return out_flat.reshape(BATCH, NCLS)


# speedup vs baseline: 1.5549x; 1.5549x over previous
import functools

import jax
import jax.numpy as jnp
from jax import lax
from jax.experimental import pallas as pl
from jax.experimental.pallas import tpu as pltpu
from jax.experimental.pallas import tpu_sc as plsc

BATCH = 16384
SEQ = 50
DIM = 64
NCLS = 2
VOCAB = 1000000

LANES = 16
NVEC = DIM // LANES

CH = 8
TOK = CH * SEQ

VBLK = 2048
NBLK = (VOCAB + VBLK - 1) // VBLK
PAD_V = NBLK * VBLK


def _relayout_body(in_ref, out_ref):
  x = in_ref[...]
  eye = jnp.float32(1.0) * (
      lax.broadcasted_iota(jnp.int32, (DIM, DIM), 0)
      == lax.broadcasted_iota(jnp.int32, (DIM, DIM), 1))
  t = lax.dot_general(x, eye, (((0,), (0,)), ((), ())),
                      precision=lax.Precision.HIGHEST,
                      preferred_element_type=jnp.float32)
  out_ref[:, 0:DIM] = t
  out_ref[:, DIM:2 * DIM] = t


@functools.cache
def _build_tc():
  return pl.pallas_call(
      _relayout_body,
      grid=(NBLK,),
      in_specs=[pl.BlockSpec((DIM, VBLK), lambda i: (0, i))],
      out_specs=pl.BlockSpec((VBLK, 2 * DIM), lambda i: (i, 0)),
      out_shape=jax.ShapeDtypeStruct((PAD_V, 2 * DIM), jnp.float32),
  )


@functools.cache
def _build_sc():
  info = plsc.get_sparse_core_info()
  nw = info.num_cores * info.num_subcores
  sent_w = BATCH // nw
  tok_w = sent_w * SEQ
  nch = sent_w // CH

  mesh = plsc.VectorSubcoreMesh(core_axis_name="c", subcore_axis_name="s")

  @functools.partial(
      pl.kernel,
      out_type=jax.ShapeDtypeStruct((BATCH * NCLS,), jnp.float32),
      mesh=mesh,
      compiler_params=pltpu.CompilerParams(
          needs_layout_passes=False, use_tc_tiling_on_sc=True),
      scratch_types=[
          pltpu.VMEM((tok_w,), jnp.int32),
          pltpu.VMEM((TOK, 2 * DIM), jnp.float32),
          pltpu.VMEM((NCLS, DIM), jnp.float32),
          pltpu.VMEM((LANES,), jnp.float32),
          pltpu.VMEM((sent_w * NCLS,), jnp.float32),
          pltpu.SemaphoreType.DMA,
      ],
  )
  def fasttext_kernel(idx_hbm, table_hbm, fcw_hbm, fcb_hbm, out_hbm,
                      idx_v, rows_v, w_v, b_v, out_v, sem):
    wid = lax.axis_index("s") * info.num_cores + lax.axis_index("c")

    pltpu.sync_copy(idx_hbm.at[pl.ds(wid * tok_w, tok_w)], idx_v)
    pltpu.sync_copy(fcw_hbm, w_v)
    pltpu.sync_copy(fcb_hbm, b_v)

    w = [[w_v[c, pl.ds(LANES * j, LANES)] for j in range(NVEC)]
         for c in range(NCLS)]
    bias_vec = b_v[pl.ds(0, LANES)]
    lane = lax.iota(jnp.int32, LANES)
    inv_seq = jnp.float32(1.0 / SEQ)

    @pl.loop(0, nch)
    def _chunk(g):
      pltpu.async_copy(
          table_hbm.at[idx_v.at[pl.ds(g * TOK, TOK)]], rows_v, sem
      ).wait()
      outvec = jnp.zeros((LANES,), jnp.float32)
      for s in range(CH):
        base = s * SEQ
        acc = tuple(rows_v[base, pl.ds(LANES * j, LANES)] for j in range(NVEC))

        def tok_body(t, carry, base=base):
          return tuple(
              carry[j] + rows_v[base + t, pl.ds(LANES * j, LANES)]
              for j in range(NVEC)
          )

        acc = lax.fori_loop(1, SEQ, tok_body, acc, unroll=7)
        for c in range(NCLS):
          prod = acc[0] * w[c][0]
          for j in range(1, NVEC):
            prod = prod + acc[j] * w[c][j]
          outvec = jnp.where(lane == (s * NCLS + c), jnp.sum(prod), outvec)
      out_v[pl.ds(g * LANES, LANES)] = outvec * inv_seq + bias_vec

    pltpu.sync_copy(out_v, out_hbm.at[pl.ds(wid * sent_w * NCLS,
                                            sent_w * NCLS)])

  return fasttext_kernel


def kernel(indices, emb_table, fc_w, fc_b):
  table_wide = _build_tc()(emb_table.T)
  idx_flat = indices.reshape(-1).astype(jnp.int32)
  fcb_tiled = jnp.tile(fc_b.astype(jnp.float32), LANES // NCLS)
  out_flat = _build_sc()(idx_flat, table_wide, fc_w, fcb_tiled)
  return out_flat.reshape(BATCH, NCLS)


# speedup vs baseline: 2.0861x; 1.3416x over previous
import functools

import jax
import jax.numpy as jnp
from jax import lax
from jax.experimental import pallas as pl
from jax.experimental.pallas import tpu as pltpu
from jax.experimental.pallas import tpu_sc as plsc

BATCH = 16384
SEQ = 50
DIM = 64
NCLS = 2
VOCAB = 1000000

LANES = 16
NVEC = DIM // LANES

CH = 8
TOK = CH * SEQ

VBLK = 8192
NBLK = (VOCAB + VBLK - 1) // VBLK
PAD_V = NBLK * VBLK


def _relayout_body(in_ref, out_ref):
  x = in_ref[...]
  eye = jnp.float32(1.0) * (
      lax.broadcasted_iota(jnp.int32, (DIM, DIM), 0)
      == lax.broadcasted_iota(jnp.int32, (DIM, DIM), 1))
  t = lax.dot_general(x, eye, (((0,), (0,)), ((), ())),
                      precision=lax.Precision.HIGHEST,
                      preferred_element_type=jnp.float32)
  out_ref[:, 0:DIM] = t
  out_ref[:, DIM:2 * DIM] = t


@functools.cache
def _build_tc():
  return pl.pallas_call(
      _relayout_body,
      grid=(NBLK,),
      in_specs=[pl.BlockSpec((DIM, VBLK), lambda i: (0, i))],
      out_specs=pl.BlockSpec((VBLK, 2 * DIM), lambda i: (i, 0)),
      out_shape=jax.ShapeDtypeStruct((PAD_V, 2 * DIM), jnp.float32),
  )


@functools.cache
def _build_sc():
  info = plsc.get_sparse_core_info()
  nw = info.num_cores * info.num_subcores
  sent_w = BATCH // nw
  tok_w = sent_w * SEQ
  nch = sent_w // CH

  mesh = plsc.VectorSubcoreMesh(core_axis_name="c", subcore_axis_name="s")

  @functools.partial(
      pl.kernel,
      out_type=jax.ShapeDtypeStruct((BATCH * NCLS,), jnp.float32),
      mesh=mesh,
      compiler_params=pltpu.CompilerParams(
          needs_layout_passes=False, use_tc_tiling_on_sc=True),
      scratch_types=[
          pltpu.VMEM((tok_w,), jnp.int32),
          pltpu.VMEM((TOK, 2 * DIM), jnp.float32),
          pltpu.VMEM((TOK, 2 * DIM), jnp.float32),
          pltpu.VMEM((NCLS, DIM), jnp.float32),
          pltpu.VMEM((LANES,), jnp.float32),
          pltpu.VMEM((sent_w * NCLS,), jnp.float32),
          pltpu.SemaphoreType.DMA,
          pltpu.SemaphoreType.DMA,
      ],
  )
  def fasttext_kernel(idx_hbm, table_hbm, fcw_hbm, fcb_hbm, out_hbm,
                      idx_v, rows_a, rows_b, w_v, b_v, out_v, sem_a, sem_b):
    wid = lax.axis_index("s") * info.num_cores + lax.axis_index("c")

    pltpu.sync_copy(idx_hbm.at[pl.ds(wid * tok_w, tok_w)], idx_v)
    pltpu.sync_copy(fcw_hbm, w_v)
    pltpu.sync_copy(fcb_hbm, b_v)

    w = [[w_v[c, pl.ds(LANES * j, LANES)] for j in range(NVEC)]
         for c in range(NCLS)]
    bias_vec = b_v[pl.ds(0, LANES)]
    lane = lax.iota(jnp.int32, LANES)
    inv_seq = jnp.float32(1.0 / SEQ)

    def issue(g, rows_v, sem):
      pltpu.async_copy(
          table_hbm.at[idx_v.at[pl.ds(g * TOK, TOK)]], rows_v, sem)

    def drain(rows_v, sem):
      pltpu.make_async_copy(
          table_hbm.at[idx_v.at[pl.ds(0, TOK)]], rows_v, sem).wait()

    def compute(g, rows_v):
      outvec = jnp.zeros((LANES,), jnp.float32)
      for s in range(CH):
        base = s * SEQ
        acc = tuple(rows_v[base, pl.ds(LANES * j, LANES)] for j in range(NVEC))

        def tok_body(t, carry, base=base, rows_v=rows_v):
          return tuple(
              carry[j] + rows_v[base + t, pl.ds(LANES * j, LANES)]
              for j in range(NVEC)
          )

        acc = lax.fori_loop(1, SEQ, tok_body, acc, unroll=7)
        for c in range(NCLS):
          prod = acc[0] * w[c][0]
          for j in range(1, NVEC):
            prod = prod + acc[j] * w[c][j]
          outvec = jnp.where(lane == (s * NCLS + c), jnp.sum(prod), outvec)
      out_v[pl.ds(g * LANES, LANES)] = outvec * inv_seq + bias_vec

    issue(0, rows_a, sem_a)
    issue(1, rows_b, sem_b)

    @pl.loop(0, nch, step=2)
    def _chunk(g):
      for k, (rows_v, sem) in enumerate(((rows_a, sem_a), (rows_b, sem_b))):
        drain(rows_v, sem)
        compute(g + k, rows_v)

        @pl.when(g + k + 2 < nch)
        def _():
          issue(g + k + 2, rows_v, sem)

    pltpu.sync_copy(out_v, out_hbm.at[pl.ds(wid * sent_w * NCLS,
                                            sent_w * NCLS)])

  return fasttext_kernel


def kernel(indices, emb_table, fc_w, fc_b):
  table_wide = _build_tc()(emb_table.T)
  idx_flat = indices.reshape(-1).astype(jnp.int32)
  fcb_tiled = jnp.tile(fc_b.astype(jnp.float32), LANES // NCLS)
  out_flat = _build_sc()(idx_flat, table_wide, fc_w, fcb_tiled)
  return out_flat.reshape(BATCH, NCLS)


# speedup vs baseline: 2.1103x; 1.0116x over previous
import functools

import jax
import jax.numpy as jnp
from jax import lax
from jax.experimental import pallas as pl
from jax.experimental.pallas import tpu as pltpu
from jax.experimental.pallas import tpu_sc as plsc

BATCH = 16384
SEQ = 50
DIM = 64
NCLS = 2
VOCAB = 1000000

LANES = 16
NVEC = DIM // LANES

CH = 8
TOK = CH * SEQ

VBLK = 10240
NBLK = (VOCAB + VBLK - 1) // VBLK
PAD_V = NBLK * VBLK


def _relayout_body(in_ref, out_ref):
  x = in_ref[...]
  eye = jnp.float32(1.0) * (
      lax.broadcasted_iota(jnp.int32, (DIM, DIM), 0)
      == lax.broadcasted_iota(jnp.int32, (DIM, DIM), 1))
  t = lax.dot_general(x, eye, (((0,), (0,)), ((), ())),
                      precision=lax.Precision.HIGHEST,
                      preferred_element_type=jnp.float32)
  out_ref[:, 0:DIM] = t
  out_ref[:, DIM:2 * DIM] = t


@functools.cache
def _build_tc():
  return pl.pallas_call(
      _relayout_body,
      grid=(NBLK,),
      in_specs=[pl.BlockSpec((DIM, VBLK), lambda i: (0, i))],
      out_specs=pl.BlockSpec((VBLK, 2 * DIM), lambda i: (i, 0)),
      out_shape=jax.ShapeDtypeStruct((PAD_V, 2 * DIM), jnp.float32),
  )


@functools.cache
def _build_sc():
  info = plsc.get_sparse_core_info()
  nw = info.num_cores * info.num_subcores
  sent_w = BATCH // nw
  tok_w = sent_w * SEQ
  nch = sent_w // CH

  mesh = plsc.VectorSubcoreMesh(core_axis_name="c", subcore_axis_name="s")

  @functools.partial(
      pl.kernel,
      out_type=jax.ShapeDtypeStruct((BATCH * NCLS,), jnp.float32),
      mesh=mesh,
      compiler_params=pltpu.CompilerParams(
          needs_layout_passes=False, use_tc_tiling_on_sc=True),
      scratch_types=[
          pltpu.VMEM((tok_w,), jnp.int32),
          pltpu.VMEM((TOK, 2 * DIM), jnp.float32),
          pltpu.VMEM((TOK, 2 * DIM), jnp.float32),
          pltpu.VMEM((NCLS, DIM), jnp.float32),
          pltpu.VMEM((LANES,), jnp.float32),
          pltpu.VMEM((sent_w * NCLS,), jnp.float32),
          pltpu.SemaphoreType.DMA,
          pltpu.SemaphoreType.DMA,
      ],
  )
  def fasttext_kernel(idx_hbm, table_hbm, fcw_hbm, fcb_hbm, out_hbm,
                      idx_v, rows_a, rows_b, w_v, b_v, out_v, sem_a, sem_b):
    wid = lax.axis_index("s") * info.num_cores + lax.axis_index("c")

    pltpu.sync_copy(idx_hbm.at[pl.ds(wid * tok_w, tok_w)], idx_v)
    pltpu.sync_copy(fcw_hbm, w_v)
    pltpu.sync_copy(fcb_hbm, b_v)

    w = [[w_v[c, pl.ds(LANES * j, LANES)] for j in range(NVEC)]
         for c in range(NCLS)]
    bias_vec = b_v[pl.ds(0, LANES)]
    lane = lax.iota(jnp.int32, LANES)
    inv_seq = jnp.float32(1.0 / SEQ)

    def issue(g, rows_v, sem):
      pltpu.async_copy(
          table_hbm.at[idx_v.at[pl.ds(g * TOK, TOK)]], rows_v, sem)

    def drain(rows_v, sem):
      pltpu.make_async_copy(
          table_hbm.at[idx_v.at[pl.ds(0, TOK)]], rows_v, sem).wait()

    def compute(g, rows_v):
      outvec = jnp.zeros((LANES,), jnp.float32)
      for s in range(CH):
        base = s * SEQ
        acc = tuple(rows_v[base, pl.ds(LANES * j, LANES)] for j in range(NVEC))

        def tok_body(t, carry, base=base, rows_v=rows_v):
          return tuple(
              carry[j] + rows_v[base + t, pl.ds(LANES * j, LANES)]
              for j in range(NVEC)
          )

        acc = lax.fori_loop(1, SEQ, tok_body, acc, unroll=7)
        for c in range(NCLS):
          prod = acc[0] * w[c][0]
          for j in range(1, NVEC):
            prod = prod + acc[j] * w[c][j]
          outvec = jnp.where(lane == (s * NCLS + c), jnp.sum(prod), outvec)
      out_v[pl.ds(g * LANES, LANES)] = outvec * inv_seq + bias_vec

    issue(0, rows_a, sem_a)
    issue(1, rows_b, sem_b)

    @pl.loop(0, nch, step=2)
    def _chunk(g):
      for k, (rows_v, sem) in enumerate(((rows_a, sem_a), (rows_b, sem_b))):
        drain(rows_v, sem)
        compute(g + k, rows_v)

        @pl.when(g + k + 2 < nch)
        def _():
          issue(g + k + 2, rows_v, sem)

    pltpu.sync_copy(out_v, out_hbm.at[pl.ds(wid * sent_w * NCLS,
                                            sent_w * NCLS)])

  return fasttext_kernel


def kernel(indices, emb_table, fc_w, fc_b):
  table_wide = _build_tc()(emb_table.T)
  idx_flat = indices.reshape(-1).astype(jnp.int32)
  fcb_tiled = jnp.tile(fc_b.astype(jnp.float32), LANES // NCLS)
  out_flat = _build_sc()(idx_flat, table_wide, fc_w, fcb_tiled)
  return out_flat.reshape(BATCH, NCLS)
